# Initial kernel scaffold; baseline (speedup 1.0000x reference)
import jax
import jax.numpy as jnp
from jax.experimental import pallas as pl


def kernel(x, edge_index, edge_attr, batch, params):
    raise NotImplementedError("write your pallas kernel here")


# speedup vs baseline: 1.0039x; 1.0039x over previous
import jax
import jax.numpy as jnp
from jax.experimental import pallas as pl
from jax.experimental.pallas import tpu as pltpu

N = 50000
NG = 1024
H = 64


def _bn(h, g, b, eps=1e-5):
    mu = jnp.mean(h, axis=0, keepdims=True)
    var = jnp.var(h, axis=0, keepdims=True)
    return (h - mu) / jnp.sqrt(var + eps) * g + b


def _final_body(pool_ref, w1_ref, b1_ref, w2_ref, b2_ref, out_ref):
    z = jnp.maximum(pool_ref[...] @ w1_ref[...] + b1_ref[...], 0.0)
    out_ref[...] = z @ w2_ref[...] + b2_ref[...]


def kernel(x, edge_index, edge_attr, batch, params):
    p = params
    src, dst = edge_index[0], edge_index[1]
    eh = edge_attr @ p["enc_w"] + p["enc_b"]
    h = x
    for l in range(3):
        e = eh @ p[f"l{l}_lin_w"] + p[f"l{l}_lin_b"]
        m = jax.nn.relu(h[src] + e)
        agg = jnp.zeros((h.shape[0], h.shape[1]), h.dtype).at[dst].add(m)
        z = h + agg
        z = _bn(z @ p[f"l{l}_w1"] + p[f"l{l}_b1"], p[f"l{l}_g1"], p[f"l{l}_be1"])
        z = jnp.maximum(z, 0.0)
        z = _bn(z @ p[f"l{l}_w2"] + p[f"l{l}_b2"], p[f"l{l}_g2"], p[f"l{l}_be2"])
        h = jnp.maximum(z, 0.0)
    sums = jax.ops.segment_sum(h, batch, num_segments=NG)
    counts = jax.ops.segment_sum(jnp.ones((h.shape[0],), h.dtype), batch, num_segments=NG)
    mean = sums / jnp.maximum(counts, 1.0)[:, None]
    pool = jnp.concatenate([mean, sums], axis=1)
    return pl.pallas_call(
        _final_body,
        out_shape=jax.ShapeDtypeStruct((NG, 128), jnp.float32),
    )(pool, p["p_w1"], p["p_b1"].reshape(1, -1), p["p_w2"], p["p_b2"].reshape(1, -1))


# speedup vs baseline: 2.4137x; 2.4043x over previous
import functools

import jax
import jax.numpy as jnp
from jax import lax
from jax.experimental import pallas as pl
from jax.experimental.pallas import tpu as pltpu
from jax.experimental.pallas import tpu_sc as plsc

N = 50000
E = 800000
NG = 1024
H = 64

NPAD = 50048
EPAD = 802816
ROWS_PER_TILE = NPAD // 16
TILE = 2000
NT = N // TILE

_f32 = jnp.float32
_i32 = jnp.int32


def _bcast16(v):
    return jnp.full((16,), v, _i32)


def _zero_rows(stage_v, width):
    def zb(i, _):
        for u in range(width // 16):
            stage_v[i, pl.ds(16 * u, 16)] = jnp.zeros((16,), _f32)
        return 0
    lax.fori_loop(0, 128, zb, 0)


def _zero_acc_slice(acc_s, stage_v, rowbase):
    def zc(j, _):
        pltpu.sync_copy(stage_v, acc_s.at[pl.ds(rowbase + j * 128, 128), :])
        return 0
    lax.fori_loop(0, 24, zc, 0)
    pltpu.sync_copy(stage_v.at[pl.ds(0, 56), :],
                    acc_s.at[pl.ds(rowbase + 24 * 128, 56), :])


def _writeout_slice(acc_s, stage_v, rowbase, out_ref):
    def wb(j, _):
        pltpu.sync_copy(acc_s.at[pl.ds(rowbase + j * 128, 128), :], stage_v)
        pltpu.sync_copy(stage_v, out_ref.at[pl.ds(rowbase + j * 128, 128), :])
        return 0
    lax.fori_loop(0, 24, wb, 0)
    pltpu.sync_copy(acc_s.at[pl.ds(rowbase + 24 * 128, 56), :],
                    stage_v.at[pl.ds(0, 56), :])
    pltpu.sync_copy(stage_v.at[pl.ds(0, 56), :],
                    out_ref.at[pl.ds(rowbase + 24 * 128, 56), :])


@functools.cache
def _sc_mesh():
    return plsc.VectorSubcoreMesh(core_axis_name="c", subcore_axis_name="s")


def _make_sc_body(feat_split):
    D = 32 if feat_split else 16
    NB = (EPAD // 16 // 128) if feat_split else (EPAD // 32 // 128)

    def body(*args):
        if feat_split:
            (h0, h1, srcp, dstp, eap, wc, agg0, agg1,
             src0, src1, dst0, dst1, ea0, ea1, hr0, hr1, m_v, w_v, acc_s,
             ss0, ss1, sd0, sd1, se0, se1, sg0, sg1) = args
            tables = (h0, h1)
        else:
            (xp, srcp, dstp, eap, wc, agg0, agg1,
             src0, src1, dst0, dst1, ea0, ea1, hr0, hr1, m_v, w_v, acc_s,
             ss0, ss1, sd0, sd1, se0, se1, sg0, sg1) = args
            tables = (xp,)
        src_v = (src0, src1)
        dst_v = (dst0, dst1)
        ea_v = (ea0, ea1)
        hrows = (hr0, hr1)
        sem_s = (ss0, ss1)
        sem_d = (sd0, sd1)
        sem_e = (se0, se1)
        sem_g = (sg0, sg1)

        c = lax.axis_index("c")
        s = lax.axis_index("s")
        pltpu.sync_copy(wc, w_v)
        rowbase = s * ROWS_PER_TILE
        _zero_rows(m_v, D)
        _zero_acc_slice(acc_s, m_v, rowbase)
        plsc.subcore_barrier()

        if feat_split:
            cs = c * 32
            bias = [w_v[6, pl.ds(cs, 16)], w_v[6, pl.ds(cs + 16, 16)]]
            wk = [[w_v[k, pl.ds(cs, 16)], w_v[k, pl.ds(cs + 16, 16)]]
                  for k in range(6)]
            ebase = s * (EPAD // 16)
        else:
            bias = [w_v[6, 0:16]]
            wk = [[w_v[k, 0:16]] for k in range(6)]
            ebase = (s * 2 + c) * (EPAD // 32)

        def start_idx(b, sl):
            eb = ebase + b * 128
            pltpu.async_copy(srcp.at[pl.ds(eb, 128)], src_v[sl], sem_s[sl])
            pltpu.async_copy(dstp.at[pl.ds(eb, 128)], dst_v[sl], sem_d[sl])
            pltpu.async_copy(eap.at[pl.ds(eb * 6, 768)],
                             ea_v[sl].at[pl.ds(0, 768)], sem_e[sl])

        def wait_idx(b, sl):
            eb = ebase + b * 128
            pltpu.make_async_copy(srcp.at[pl.ds(eb, 128)], src_v[sl],
                                  sem_s[sl]).wait()
            pltpu.make_async_copy(dstp.at[pl.ds(eb, 128)], dst_v[sl],
                                  sem_d[sl]).wait()
            pltpu.make_async_copy(eap.at[pl.ds(eb * 6, 768)],
                                  ea_v[sl].at[pl.ds(0, 768)],
                                  sem_e[sl]).wait()

        def start_gather(sl):
            if feat_split:
                @pl.when(c == 0)
                def _():
                    pltpu.async_copy(tables[0].at[src_v[sl]], hrows[sl],
                                     sem_g[sl])

                @pl.when(c == 1)
                def _():
                    pltpu.async_copy(tables[1].at[src_v[sl]], hrows[sl],
                                     sem_g[sl])
            else:
                pltpu.async_copy(tables[0].at[src_v[sl]], hrows[sl],
                                 sem_g[sl])

        def wait_gather(sl):
            if feat_split:
                @pl.when(c == 0)
                def _():
                    pltpu.make_async_copy(tables[0].at[src_v[sl]], hrows[sl],
                                          sem_g[sl]).wait()

                @pl.when(c == 1)
                def _():
                    pltpu.make_async_copy(tables[1].at[src_v[sl]], hrows[sl],
                                          sem_g[sl]).wait()
            else:
                pltpu.make_async_copy(tables[0].at[src_v[sl]], hrows[sl],
                                      sem_g[sl]).wait()

        def compute(sl):
            hr = hrows[sl]
            ev = ea_v[sl]

            kidx = lax.iota(_i32, 16)

            def edge_body(i2, _):
                for e in range(2):
                    i = i2 * 2 + e
                    ii = _bcast16(i) * 6
                    attrs = plsc.load_gather(ev, [ii + kidx])
                    accs = list(bias)
                    for k in range(6):
                        sk = attrs.at[_bcast16(k)].get(
                            mode="promise_in_bounds")
                        for u in range(len(accs)):
                            accs[u] = accs[u] + sk * wk[k][u]
                    for u in range(len(accs)):
                        m_v[i, pl.ds(16 * u, 16)] = jnp.maximum(
                            accs[u] + hr[i, pl.ds(16 * u, 16)], 0.0)
                return 0
            lax.fori_loop(0, 64, edge_body, 0)

        nbr = jnp.minimum(NB, (E - ebase) // 128)

        start_idx(0, 0)
        start_idx(1, 1)
        wait_idx(0, 0)
        start_gather(0)

        def pair_body(b2, _):
            for j in (0, 1):
                b = b2 * 2 + j
                o = 1 - j

                @pl.when(b + 1 < nbr)
                def _():
                    wait_idx(b + 1, o)
                    start_gather(o)
                wait_gather(j)
                compute(j)
                pltpu.sync_copy(m_v, acc_s.at[dst_v[j]], add=True)

                @pl.when(b + 2 < nbr)
                def _():
                    start_idx(b + 2, j)
            return 0
        lax.fori_loop(0, nbr // 2, pair_body, 0)
        plsc.subcore_barrier()

        @pl.when(c == 0)
        def _():
            _writeout_slice(acc_s, m_v, rowbase, agg0)

        @pl.when(c == 1)
        def _():
            _writeout_slice(acc_s, m_v, rowbase, agg1)

    return body


def _sc_scratch(D, wcols):
    return [
        pltpu.VMEM((128,), _i32), pltpu.VMEM((128,), _i32),
        pltpu.VMEM((128,), _i32), pltpu.VMEM((128,), _i32),
        pltpu.VMEM((784,), _f32), pltpu.VMEM((784,), _f32),
        pltpu.VMEM((128, D), _f32), pltpu.VMEM((128, D), _f32),
        pltpu.VMEM((128, D), _f32),
        pltpu.VMEM((8, wcols), _f32),
        pltpu.VMEM_SHARED((NPAD, D), _f32),
    ] + [pltpu.SemaphoreType.DMA] * 8


@functools.cache
def _sc_edge_l0_kernel():
    return pl.kernel(
        _make_sc_body(False),
        out_type=(jax.ShapeDtypeStruct((NPAD, 16), _f32),
                  jax.ShapeDtypeStruct((NPAD, 16), _f32)),
        mesh=_sc_mesh(),
        compiler_params=pltpu.CompilerParams(needs_layout_passes=False,
                                             use_tc_tiling_on_sc=False),
        scratch_types=_sc_scratch(16, 16),
    )


def _sc_edge_l0(*args):
    return _sc_edge_l0_kernel()(*args)


@functools.cache
def _sc_edge_feat_kernel():
    return pl.kernel(
        _make_sc_body(True),
        out_type=(jax.ShapeDtypeStruct((NPAD, 32), _f32),
                  jax.ShapeDtypeStruct((NPAD, 32), _f32)),
        mesh=_sc_mesh(),
        compiler_params=pltpu.CompilerParams(needs_layout_passes=False,
                                             use_tc_tiling_on_sc=False),
        scratch_types=_sc_scratch(32, 64),
    )


def _sc_edge_feat(*args):
    return _sc_edge_feat_kernel()(*args)


def _fold(saug, dp, w, b, g, be):
    s2 = saug[:dp, :dp] / N
    mu_z = saug[dp:dp + 1, :dp] / N
    zw_mu = jnp.dot(mu_z, w, precision=lax.Precision.HIGHEST)
    mu1 = zw_mu + b
    sw = jnp.dot(s2, w, precision=lax.Precision.HIGHEST)
    e2 = jnp.sum(w * sw, axis=0, keepdims=True) + 2.0 * b * zw_mu + b * b
    var = e2 - mu1 * mu1
    inv = g / jnp.sqrt(var + 1e-5)
    return w * inv, b * inv + be - mu1 * inv


def make_fused_layer(kind):
    dp = 16 if kind == "l0" else 64
    n_in_feat = 3 if kind == "l0" else 4

    def body(*refs):
        nf = 3 if kind == "l0" else 4
        if kind == "l0":
            xp_ref, a0_ref, a1_ref = refs[:3]
        else:
            h0_ref, h1_ref, a0_ref, a1_ref = refs[:4]
        (w1_ref, b1_ref, g1_ref, be1_ref, w2_ref, b2_ref, g2_ref,
         be2_ref) = refs[nf:nf + 8]
        rest = refs[nf + 8:]
        if kind == "last":
            batch_ref, pw1_ref, pb1_ref, pw2_ref, pb2_ref = rest[:5]
            out_ref = rest[5]
            z_s, saug_s, raug_s, w1f_s, b1f_s, w2f_s, b2f_s, psum_s, cnt_s = rest[6:]
        else:
            out0_ref, out1_ref = rest[:2]
            z_s, saug_s, raug_s, w1f_s, b1f_s, w2f_s, b2f_s = rest[2:]

        ph = pl.program_id(0)
        i = pl.program_id(1)
        rows = pl.ds(i * TILE, TILE)

        @pl.when(jnp.logical_and(ph == 0, i == 0))
        def _():
            saug_s[...] = jnp.zeros_like(saug_s)
            raug_s[...] = jnp.zeros_like(raug_s)
            if kind == "last":
                psum_s[...] = jnp.zeros_like(psum_s)
                cnt_s[...] = jnp.zeros_like(cnt_s)

        @pl.when(ph == 0)
        def _():
            if kind == "l0":
                zt = xp_ref[...] + a0_ref[...] + a1_ref[...]
            else:
                zt = jnp.concatenate([h0_ref[...] + a0_ref[...],
                                      h1_ref[...] + a1_ref[...]], axis=1)
            z_s[rows, :] = zt
            za = jnp.concatenate([zt, jnp.ones((TILE, 1), _f32)], axis=1)
            saug_s[...] += lax.dot_general(za, za, (((0,), (0,)), ((), ())),
                                           precision=lax.Precision.HIGHEST,
                                           preferred_element_type=_f32)

        @pl.when(jnp.logical_and(ph == 1, i == 0))
        def _():
            w1f, b1f = _fold(saug_s[...], dp, w1_ref[...], b1_ref[...],
                             g1_ref[...], be1_ref[...])
            w1f_s[...] = w1f
            b1f_s[...] = b1f

        @pl.when(ph == 1)
        def _():
            r = jnp.maximum(jnp.dot(z_s[rows, :], w1f_s[...],
                                    precision=lax.Precision.HIGHEST)
                            + b1f_s[...], 0.0)
            ra = jnp.concatenate([r, jnp.ones((TILE, 1), _f32)], axis=1)
            raug_s[...] += lax.dot_general(ra, ra, (((0,), (0,)), ((), ())),
                                           precision=lax.Precision.HIGHEST,
                                           preferred_element_type=_f32)

        @pl.when(jnp.logical_and(ph == 2, i == 0))
        def _():
            w2f, b2f = _fold(raug_s[...], 2 * H, w2_ref[...], b2_ref[...],
                             g2_ref[...], be2_ref[...])
            w2f_s[...] = w2f
            b2f_s[...] = b2f

        @pl.when(ph == 2)
        def _():
            r = jnp.maximum(jnp.dot(z_s[rows, :], w1f_s[...],
                                    precision=lax.Precision.HIGHEST)
                            + b1f_s[...], 0.0)
            h = jnp.maximum(jnp.dot(r, w2f_s[...],
                                    precision=lax.Precision.HIGHEST)
                            + b2f_s[...], 0.0)
            if kind == "last":
                bt = batch_ref[0, 0, :]
                oh = (lax.broadcasted_iota(_i32, (NG, TILE), 0)
                      == bt[None, :]).astype(_f32)
                psum_s[...] += lax.dot_general(oh, h, (((1,), (0,)), ((), ())),
                                               precision=lax.Precision.HIGHEST,
                                               preferred_element_type=_f32)
                cnt_s[...] += jnp.sum(oh, axis=1, keepdims=True)

                @pl.when(i == NT - 1)
                def _():
                    sums = psum_s[...]
                    mean = sums / jnp.maximum(cnt_s[...], 1.0)
                    pool = jnp.concatenate([mean, sums], axis=1)
                    zz = jnp.maximum(
                        jnp.dot(pool, pw1_ref[...],
                                precision=lax.Precision.HIGHEST)
                        + pb1_ref[...], 0.0)
                    out_ref[...] = jnp.dot(
                        zz, pw2_ref[...],
                        precision=lax.Precision.HIGHEST) + pb2_ref[...]
            else:
                out0_ref[...] = h[:, :32]
                out1_ref[...] = h[:, 32:]

    def row_spec(w):
        return pl.BlockSpec((TILE, w),
                            lambda ph, i: (jnp.where(ph == 0, i, 0), 0))

    def const_spec(shape):
        nd = len(shape)
        return pl.BlockSpec(shape, lambda ph, i, _nd=nd: (0,) * _nd)

    if kind == "l0":
        in_specs = [row_spec(16), row_spec(16), row_spec(16)]
    else:
        in_specs = [row_spec(32)] * 4
    in_specs += [const_spec((dp, 2 * H)), const_spec((1, 2 * H)),
                 const_spec((1, 2 * H)), const_spec((1, 2 * H)),
                 const_spec((2 * H, H)), const_spec((1, H)),
                 const_spec((1, H)), const_spec((1, H))]
    scratch = [pltpu.VMEM((N, dp), _f32),
               pltpu.VMEM((dp + 1, dp + 1), _f32),
               pltpu.VMEM((2 * H + 1, 2 * H + 1), _f32),
               pltpu.VMEM((dp, 2 * H), _f32),
               pltpu.VMEM((1, 2 * H), _f32),
               pltpu.VMEM((2 * H, H), _f32),
               pltpu.VMEM((1, H), _f32)]
    if kind == "last":
        in_specs += [pl.BlockSpec((1, 1, TILE), lambda ph, i: (i, 0, 0)),
                     const_spec((2 * H, 128)), const_spec((1, 128)),
                     const_spec((128, 128)), const_spec((1, 128))]
        out_specs = const_spec((NG, 128))
        out_shape = jax.ShapeDtypeStruct((NG, 128), _f32)
        scratch = scratch + [pltpu.VMEM((NG, H), _f32),
                             pltpu.VMEM((NG, 1), _f32)]
    else:
        out_specs = [pl.BlockSpec((TILE, 32),
                                  lambda ph, i: (jnp.where(ph == 2, i, 0), 0))] * 2
        out_shape = [jax.ShapeDtypeStruct((N, 32), _f32),
                     jax.ShapeDtypeStruct((N, 32), _f32)]

    return pl.pallas_call(
        body,
        grid=(3, NT),
        in_specs=in_specs,
        out_specs=out_specs,
        out_shape=out_shape,
        scratch_shapes=scratch,
    )


def kernel(x, edge_index, edge_attr, batch, params):
    with jax.default_matmul_precision("highest"):
        return _kernel_impl(x, edge_index, edge_attr, batch, params)


def _kernel_impl(x, edge_index, edge_attr, batch, params):
    p = params
    src = edge_index[0]
    dst = edge_index[1]
    eap = edge_attr.reshape(-1)
    xpad = jnp.pad(x, ((0, 0), (0, 5)))
    batch3 = batch.reshape(NT, 1, TILE)

    def wc_for(l, dcols):
        w = p["enc_w"] @ p[f"l{l}_lin_w"]
        c = p["enc_b"] @ p[f"l{l}_lin_w"] + p[f"l{l}_lin_b"]
        wc = jnp.zeros((8, dcols), _f32)
        wc = wc.at[:6, :w.shape[1]].set(w)
        wc = wc.at[6, :w.shape[1]].set(c)
        return wc

    def prms(l):
        return (p[f"l{l}_w1"] if l else jnp.pad(p["l0_w1"], ((0, 5), (0, 0))),
                p[f"l{l}_b1"].reshape(1, -1), p[f"l{l}_g1"].reshape(1, -1),
                p[f"l{l}_be1"].reshape(1, -1), p[f"l{l}_w2"],
                p[f"l{l}_b2"].reshape(1, -1), p[f"l{l}_g2"].reshape(1, -1),
                p[f"l{l}_be2"].reshape(1, -1))

    a0, a1 = _sc_edge_l0(xpad, src, dst, eap, wc_for(0, 16))
    h0, h1 = make_fused_layer("l0")(xpad, a0, a1, *prms(0))

    a0, a1 = _sc_edge_feat(h0, h1, src, dst, eap, wc_for(1, 64))
    h0, h1 = make_fused_layer("mid")(h0, h1, a0, a1, *prms(1))

    a0, a1 = _sc_edge_feat(h0, h1, src, dst, eap, wc_for(2, 64))
    return make_fused_layer("last")(h0, h1, a0, a1, *prms(2), batch3,
                                    p["p_w1"], p["p_b1"].reshape(1, -1),
                                    p["p_w2"], p["p_b2"].reshape(1, -1))
